# Initial kernel scaffold; baseline (speedup 1.0000x reference)
import jax
import jax.numpy as jnp
from jax.experimental import pallas as pl


def kernel(z, embedding):
    raise NotImplementedError("write your pallas kernel here")


# speedup vs baseline: 2.4799x; 2.4799x over previous
import jax
import jax.numpy as jnp
from jax.experimental import pallas as pl

EMBED_DIM = 32
N_EMBED = 512
BLOCK_T = 2048


def _vq_kernel(z_ref, emb_ref, embt_ref, zq_ref, idx_ref, loss_ref):
    i = pl.program_id(0)

    et = embt_ref[...]
    cn2 = jnp.sum(et * et, axis=0, keepdims=True)
    ent = et / jnp.maximum(jnp.sqrt(cn2), 1e-12)

    e = emb_ref[...]
    en = e / jnp.maximum(
        jnp.sqrt(jnp.sum(e * e, axis=1, keepdims=True)), 1e-12
    )

    zb = z_ref[...]
    zn = zb / jnp.maximum(
        jnp.sqrt(jnp.sum(zb * zb, axis=1, keepdims=True)), 1e-12
    )

    s = jnp.dot(zn, ent, preferred_element_type=jnp.float32)
    en2 = jnp.sum(ent * ent, axis=0, keepdims=True)
    d2 = en2 - 2.0 * s

    m = jnp.min(d2, axis=1, keepdims=True)
    lane = jax.lax.broadcasted_iota(jnp.int32, d2.shape, 1)
    idxc = jnp.min(
        jnp.where(d2 == m, lane, N_EMBED), axis=1, keepdims=True
    )

    onehot = (lane == idxc).astype(jnp.float32)
    zq = jnp.dot(onehot, en, preferred_element_type=jnp.float32)

    zq_ref[...] = zb + (zq - zb)
    idx_ref[...] = idxc

    diff = zq - zn
    part = jnp.sum(
        jnp.sum(diff * diff, axis=1, keepdims=True), axis=0, keepdims=True
    )

    @pl.when(i == 0)
    def _():
        loss_ref[...] = jnp.zeros((1, 1), jnp.float32)

    loss_ref[...] += part


@jax.jit
def kernel(z, embedding):
    B, T, D = z.shape
    n_tok = B * T
    flat = z.reshape(n_tok, D)
    grid = n_tok // BLOCK_T

    zq, idxcol, loss_sum = pl.pallas_call(
        _vq_kernel,
        grid=(grid,),
        in_specs=[
            pl.BlockSpec((BLOCK_T, D), lambda i: (i, 0)),
            pl.BlockSpec((N_EMBED, D), lambda i: (0, 0)),
            pl.BlockSpec((D, N_EMBED), lambda i: (0, 0)),
        ],
        out_specs=[
            pl.BlockSpec((BLOCK_T, D), lambda i: (i, 0)),
            pl.BlockSpec((BLOCK_T, 1), lambda i: (i, 0)),
            pl.BlockSpec((1, 1), lambda i: (0, 0)),
        ],
        out_shape=[
            jax.ShapeDtypeStruct((n_tok, D), jnp.float32),
            jax.ShapeDtypeStruct((n_tok, 1), jnp.int32),
            jax.ShapeDtypeStruct((1, 1), jnp.float32),
        ],
    )(flat, embedding, embedding.T)

    z_q = zq.reshape(B, T, D)
    encoding_indices = idxcol.reshape(B, T)
    loss = 2.0 * loss_sum[0, 0] / (n_tok * D)
    return z_q, loss, encoding_indices


# speedup vs baseline: 2.9726x; 1.1987x over previous
import jax
import jax.numpy as jnp
from jax.experimental import pallas as pl
from jax.experimental.pallas import tpu as pltpu

EMBED_DIM = 32
N_EMBED = 512
BLOCK_T = 2048


def _vq_kernel(z_ref, emb_ref, embt_ref, zq_ref, idx_ref, loss_ref,
               aug_ref, en_ref):
    i = pl.program_id(0)

    @pl.when(i == 0)
    def _():
        et = embt_ref[...]
        cn2 = jnp.sum(et * et, axis=0, keepdims=True)
        cinv = 1.0 / jnp.maximum(jnp.sqrt(cn2), 1e-12)
        ent = et * cinv
        en2 = cn2 * (cinv * cinv)
        aug_ref[...] = jnp.concatenate([ent * -2.0, en2], axis=0)
        e = emb_ref[...]
        rinv = 1.0 / jnp.maximum(
            jnp.sqrt(jnp.sum(e * e, axis=1, keepdims=True)), 1e-12
        )
        en_ref[...] = e * rinv

    zb = z_ref[...]
    zinv = 1.0 / jnp.maximum(
        jnp.sqrt(jnp.sum(zb * zb, axis=1, keepdims=True)), 1e-12
    )
    zn = zb * zinv
    zna = jnp.concatenate(
        [zn, jnp.ones((BLOCK_T, 1), jnp.float32)], axis=1
    )

    d2 = jnp.dot(zna, aug_ref[...], preferred_element_type=jnp.float32)

    m = jnp.min(d2, axis=1, keepdims=True)
    lanef = jax.lax.broadcasted_iota(jnp.int32, d2.shape, 1).astype(
        jnp.float32
    )
    idxf = jnp.min(
        jnp.where(d2 == m, lanef, float(N_EMBED)), axis=1, keepdims=True
    )

    onehot = (lanef == idxf).astype(jnp.float32)
    zq = jnp.dot(onehot, en_ref[...], preferred_element_type=jnp.float32)

    zq_ref[...] = zb + (zq - zb)
    idx_ref[...] = idxf.astype(jnp.int32)

    diff = zq - zn
    part = jnp.sum(
        jnp.sum(diff * diff, axis=1, keepdims=True), axis=0, keepdims=True
    )

    @pl.when(i == 0)
    def _():
        loss_ref[...] = jnp.zeros((1, 1), jnp.float32)

    loss_ref[...] += part


@jax.jit
def kernel(z, embedding):
    B, T, D = z.shape
    n_tok = B * T
    flat = z.reshape(n_tok, D)
    grid = n_tok // BLOCK_T

    zq, idxcol, loss_sum = pl.pallas_call(
        _vq_kernel,
        grid=(grid,),
        in_specs=[
            pl.BlockSpec((BLOCK_T, D), lambda i: (i, 0)),
            pl.BlockSpec((N_EMBED, D), lambda i: (0, 0)),
            pl.BlockSpec((D, N_EMBED), lambda i: (0, 0)),
        ],
        out_specs=[
            pl.BlockSpec((BLOCK_T, D), lambda i: (i, 0)),
            pl.BlockSpec((BLOCK_T, 1), lambda i: (i, 0)),
            pl.BlockSpec((1, 1), lambda i: (0, 0)),
        ],
        out_shape=[
            jax.ShapeDtypeStruct((n_tok, D), jnp.float32),
            jax.ShapeDtypeStruct((n_tok, 1), jnp.int32),
            jax.ShapeDtypeStruct((1, 1), jnp.float32),
        ],
        scratch_shapes=[
            pltpu.VMEM((EMBED_DIM + 1, N_EMBED), jnp.float32),
            pltpu.VMEM((N_EMBED, EMBED_DIM), jnp.float32),
        ],
    )(flat, embedding, embedding.T)

    z_q = zq.reshape(B, T, D)
    encoding_indices = idxcol.reshape(B, T)
    loss = 2.0 * loss_sum[0, 0] / (n_tok * D)
    return z_q, loss, encoding_indices


# speedup vs baseline: 3.4652x; 1.1657x over previous
import jax
import jax.numpy as jnp
from jax.experimental import pallas as pl
from jax.experimental.pallas import tpu as pltpu

EMBED_DIM = 32
N_EMBED = 512
BLOCK_T = 2048


def _vq_kernel(z_ref, emb_ref, zq_ref, idx_ref, loss_ref, aug_ref, en_ref):
    i = pl.program_id(0)

    @pl.when(i == 0)
    def _():
        e = emb_ref[...]
        rn2 = jnp.sum(e * e, axis=1, keepdims=True)
        rinv = jax.lax.rsqrt(jnp.maximum(rn2, 1e-24))
        en = e * rinv
        en_ref[...] = en
        en2c = rn2 * (rinv * rinv)
        aug_ref[...] = jnp.swapaxes(
            jnp.concatenate([en * -2.0, en2c], axis=1), 0, 1
        )

    zb = z_ref[...]
    ss = jnp.sum(zb * zb, axis=1, keepdims=True)
    zinv = jax.lax.rsqrt(jnp.maximum(ss, 1e-24))
    zn = zb * zinv
    zna = jnp.concatenate(
        [zn, jnp.ones((BLOCK_T, 1), jnp.float32)], axis=1
    )

    d2 = jnp.dot(zna, aug_ref[...], preferred_element_type=jnp.float32)

    m = jnp.min(d2, axis=1, keepdims=True)
    lanef = jax.lax.broadcasted_iota(jnp.int32, d2.shape, 1).astype(
        jnp.float32
    )
    idxf = jnp.min(
        jnp.where(d2 == m, lanef, float(N_EMBED)), axis=1, keepdims=True
    )

    onehot = (lanef == idxf).astype(jnp.float32)
    zq = jnp.dot(onehot, en_ref[...], preferred_element_type=jnp.float32)

    zq_ref[...] = zb + (zq - zb)
    idx_ref[...] = (
        jnp.swapaxes(idxf, 0, 1).astype(jnp.int32).reshape(1, 1, BLOCK_T)
    )

    part = jnp.sum(ss * (zinv * zinv) + m, axis=0, keepdims=True)

    @pl.when(i == 0)
    def _():
        loss_ref[...] = jnp.zeros((1, 1), jnp.float32)

    loss_ref[...] += part


@jax.jit
def kernel(z, embedding):
    B, T, D = z.shape
    n_tok = B * T
    flat = z.reshape(n_tok, D)
    grid = n_tok // BLOCK_T

    zq, idx3, loss_sum = pl.pallas_call(
        _vq_kernel,
        grid=(grid,),
        in_specs=[
            pl.BlockSpec((BLOCK_T, D), lambda i: (i, 0)),
            pl.BlockSpec((N_EMBED, D), lambda i: (0, 0)),
        ],
        out_specs=[
            pl.BlockSpec((BLOCK_T, D), lambda i: (i, 0)),
            pl.BlockSpec((1, 1, BLOCK_T), lambda i: (i, 0, 0)),
            pl.BlockSpec((1, 1), lambda i: (0, 0)),
        ],
        out_shape=[
            jax.ShapeDtypeStruct((n_tok, D), jnp.float32),
            jax.ShapeDtypeStruct((grid, 1, BLOCK_T), jnp.int32),
            jax.ShapeDtypeStruct((1, 1), jnp.float32),
        ],
        scratch_shapes=[
            pltpu.VMEM((EMBED_DIM + 1, N_EMBED), jnp.float32),
            pltpu.VMEM((N_EMBED, EMBED_DIM), jnp.float32),
        ],
    )(flat, embedding)

    z_q = zq.reshape(B, T, D)
    encoding_indices = idx3.reshape(B, T)
    loss = 2.0 * loss_sum[0, 0] / (n_tok * D)
    return z_q, loss, encoding_indices


# speedup vs baseline: 4.7030x; 1.3572x over previous
import jax
import jax.numpy as jnp
from jax.experimental import pallas as pl
from jax.experimental.pallas import tpu as pltpu

EMBED_DIM = 32
N_EMBED = 512
BLOCK_T = 1024


def _vq_kernel(zt_ref, emb_ref, zqt_ref, idx_ref, loss_ref,
               aug_ref, ent_ref):
    i = pl.program_id(0)

    @pl.when(i == 0)
    def _():
        e = emb_ref[...]
        rn2 = jnp.sum(e * e, axis=1, keepdims=True)
        rinv = jax.lax.rsqrt(jnp.maximum(rn2, 1e-24))
        en = e * rinv
        en2c = rn2 * (rinv * rinv)
        aug_ref[...] = jnp.concatenate([en * -2.0, en2c], axis=1)
        ent_ref[...] = jnp.swapaxes(en, 0, 1)

    zbt = zt_ref[...]
    ss = jnp.sum(zbt * zbt, axis=0, keepdims=True)
    zinv = jax.lax.rsqrt(jnp.maximum(ss, 1e-24))
    znt = zbt * zinv
    zna = jnp.concatenate(
        [znt, jnp.ones((1, BLOCK_T), jnp.float32)], axis=0
    )

    d2 = jnp.dot(aug_ref[...], zna, preferred_element_type=jnp.float32)

    m = jnp.min(d2, axis=0, keepdims=True)
    rowf = jax.lax.broadcasted_iota(jnp.int32, d2.shape, 0).astype(
        jnp.float32
    )
    idxf = jnp.min(
        jnp.where(d2 == m, rowf, float(N_EMBED)), axis=0, keepdims=True
    )

    onehot = (rowf == idxf).astype(jnp.float32)
    zqt = jnp.dot(
        ent_ref[...], onehot, preferred_element_type=jnp.float32
    )

    zqt_ref[...] = zbt + (zqt - zbt)
    idx_ref[...] = idxf.astype(jnp.int32).reshape(1, 1, BLOCK_T)

    part = jnp.sum(
        ss * (zinv * zinv) + m, axis=1, keepdims=True
    )

    @pl.when(i == 0)
    def _():
        loss_ref[...] = jnp.zeros((1, 1), jnp.float32)

    loss_ref[...] += part


@jax.jit
def kernel(z, embedding):
    B, T, D = z.shape
    n_tok = B * T
    zt = jnp.transpose(z, (0, 2, 1)).reshape(B * D, T)
    grid = n_tok // BLOCK_T

    zqt, idx3, loss_sum = pl.pallas_call(
        _vq_kernel,
        grid=(grid,),
        in_specs=[
            pl.BlockSpec((D, BLOCK_T), lambda i: (i, 0)),
            pl.BlockSpec((N_EMBED, D), lambda i: (0, 0)),
        ],
        out_specs=[
            pl.BlockSpec((D, BLOCK_T), lambda i: (i, 0)),
            pl.BlockSpec((1, 1, BLOCK_T), lambda i: (i, 0, 0)),
            pl.BlockSpec((1, 1), lambda i: (0, 0)),
        ],
        out_shape=[
            jax.ShapeDtypeStruct((B * D, T), jnp.float32),
            jax.ShapeDtypeStruct((grid, 1, BLOCK_T), jnp.int32),
            jax.ShapeDtypeStruct((1, 1), jnp.float32),
        ],
        scratch_shapes=[
            pltpu.VMEM((N_EMBED, EMBED_DIM + 1), jnp.float32),
            pltpu.VMEM((EMBED_DIM, N_EMBED), jnp.float32),
        ],
    )(zt, embedding)

    z_q = jnp.transpose(zqt.reshape(B, D, T), (0, 2, 1))
    encoding_indices = idx3.reshape(B, T)
    loss = 2.0 * loss_sum[0, 0] / (n_tok * D)
    return z_q, loss, encoding_indices


# speedup vs baseline: 5.3159x; 1.1303x over previous
import jax
import jax.numpy as jnp
from jax.experimental import pallas as pl
from jax.experimental.pallas import tpu as pltpu

EMBED_DIM = 32
N_EMBED = 512
BLOCK_T = 1024
HALVES = 2


def _vq_half(zbt, aug, ent):
    ss = jnp.sum(zbt * zbt, axis=0, keepdims=True)
    zinv = jax.lax.rsqrt(jnp.maximum(ss, 1e-24))
    znt = zbt * zinv
    zna = jnp.concatenate(
        [znt, jnp.ones((1, BLOCK_T), jnp.float32)], axis=0
    )

    d2 = jnp.dot(aug, zna, preferred_element_type=jnp.float32)

    m = jnp.min(d2, axis=0, keepdims=True)
    rowf = jax.lax.broadcasted_iota(jnp.int32, d2.shape, 0).astype(
        jnp.float32
    )
    idxf = jnp.min(
        jnp.where(d2 == m, rowf, float(N_EMBED)), axis=0, keepdims=True
    )

    onehot = (rowf == idxf).astype(jnp.float32)
    zqt = jnp.dot(ent, onehot, preferred_element_type=jnp.float32)

    part = jnp.sum(ss * (zinv * zinv) + m, axis=1, keepdims=True)
    return zbt + (zqt - zbt), idxf.astype(jnp.int32), part


def _vq_kernel(zt_ref, emb_ref, zqt_ref, idx_ref, loss_ref,
               aug_ref, ent_ref):
    i = pl.program_id(0)

    @pl.when(i == 0)
    def _():
        e = emb_ref[...]
        rn2 = jnp.sum(e * e, axis=1, keepdims=True)
        rinv = jax.lax.rsqrt(jnp.maximum(rn2, 1e-24))
        en = e * rinv
        en2c = rn2 * (rinv * rinv)
        aug_ref[...] = jnp.concatenate([en * -2.0, en2c], axis=1)
        ent_ref[...] = jnp.swapaxes(en, 0, 1)

    aug = aug_ref[...]
    ent = ent_ref[...]
    zbt = zt_ref[...]

    part_sum = jnp.zeros((1, 1), jnp.float32)
    zq_halves = []
    idx_halves = []
    for h in range(HALVES):
        zq_h, idx_h, part = _vq_half(
            zbt[h * EMBED_DIM:(h + 1) * EMBED_DIM, :], aug, ent
        )
        zq_halves.append(zq_h)
        idx_halves.append(idx_h.reshape(1, 1, BLOCK_T))
        part_sum = part_sum + part

    zqt_ref[...] = jnp.concatenate(zq_halves, axis=0)
    idx_ref[...] = jnp.concatenate(idx_halves, axis=1)

    @pl.when(i == 0)
    def _():
        loss_ref[...] = jnp.zeros((1, 1), jnp.float32)

    loss_ref[...] += part_sum


@jax.jit
def kernel(z, embedding):
    B, T, D = z.shape
    n_tok = B * T
    zt = jnp.transpose(z, (0, 2, 1)).reshape(B * D, T)
    grid = B // HALVES

    zqt, idx3, loss_sum = pl.pallas_call(
        _vq_kernel,
        grid=(grid,),
        in_specs=[
            pl.BlockSpec((HALVES * D, BLOCK_T), lambda i: (i, 0)),
            pl.BlockSpec((N_EMBED, D), lambda i: (0, 0)),
        ],
        out_specs=[
            pl.BlockSpec((HALVES * D, BLOCK_T), lambda i: (i, 0)),
            pl.BlockSpec((1, HALVES, BLOCK_T), lambda i: (i, 0, 0)),
            pl.BlockSpec((1, 1), lambda i: (0, 0)),
        ],
        out_shape=[
            jax.ShapeDtypeStruct((B * D, T), jnp.float32),
            jax.ShapeDtypeStruct((grid, HALVES, BLOCK_T), jnp.int32),
            jax.ShapeDtypeStruct((1, 1), jnp.float32),
        ],
        scratch_shapes=[
            pltpu.VMEM((N_EMBED, EMBED_DIM + 1), jnp.float32),
            pltpu.VMEM((EMBED_DIM, N_EMBED), jnp.float32),
        ],
    )(zt, embedding)

    z_q = jnp.transpose(zqt.reshape(B, D, T), (0, 2, 1))
    encoding_indices = idx3.reshape(B, T)
    loss = 2.0 * loss_sum[0, 0] / (n_tok * D)
    return z_q, loss, encoding_indices


# speedup vs baseline: 5.7074x; 1.0736x over previous
import jax
import jax.numpy as jnp
from jax.experimental import pallas as pl
from jax.experimental.pallas import tpu as pltpu

EMBED_DIM = 32
N_EMBED = 512
BLOCK_T = 1024
HALVES = 4


def _vq_half(zbt, aug, ent):
    ss = jnp.sum(zbt * zbt, axis=0, keepdims=True)
    zinv = jax.lax.rsqrt(jnp.maximum(ss, 1e-24))
    znt = zbt * zinv
    zna = jnp.concatenate(
        [znt, jnp.ones((1, BLOCK_T), jnp.float32)], axis=0
    )

    d2 = jnp.dot(aug, zna, preferred_element_type=jnp.float32)

    m = jnp.min(d2, axis=0, keepdims=True)
    rowf = jax.lax.broadcasted_iota(jnp.int32, d2.shape, 0).astype(
        jnp.float32
    )
    idxf = jnp.min(
        jnp.where(d2 == m, rowf, float(N_EMBED)), axis=0, keepdims=True
    )

    onehot = (rowf == idxf).astype(jnp.float32)
    zqt = jnp.dot(ent, onehot, preferred_element_type=jnp.float32)

    part = jnp.sum(ss * (zinv * zinv) + m, axis=1, keepdims=True)
    return zbt + (zqt - zbt), idxf.astype(jnp.int32), part


def _vq_kernel(zt_ref, emb_ref, zqt_ref, idx_ref, loss_ref,
               aug_ref, ent_ref):
    i = pl.program_id(0)

    @pl.when(i == 0)
    def _():
        e = emb_ref[...]
        rn2 = jnp.sum(e * e, axis=1, keepdims=True)
        rinv = jax.lax.rsqrt(jnp.maximum(rn2, 1e-24))
        en = e * rinv
        en2c = rn2 * (rinv * rinv)
        aug_ref[...] = jnp.concatenate([en * -2.0, en2c], axis=1)
        ent_ref[...] = jnp.swapaxes(en, 0, 1)

    aug = aug_ref[...]
    ent = ent_ref[...]
    zbt = zt_ref[...]

    part_sum = jnp.zeros((1, 1), jnp.float32)
    zq_halves = []
    idx_halves = []
    for h in range(HALVES):
        zq_h, idx_h, part = _vq_half(
            zbt[h * EMBED_DIM:(h + 1) * EMBED_DIM, :], aug, ent
        )
        zq_halves.append(zq_h)
        idx_halves.append(idx_h.reshape(1, 1, BLOCK_T))
        part_sum = part_sum + part

    zqt_ref[...] = jnp.concatenate(zq_halves, axis=0)
    idx_ref[...] = jnp.concatenate(idx_halves, axis=1)

    @pl.when(i == 0)
    def _():
        loss_ref[...] = jnp.zeros((1, 1), jnp.float32)

    loss_ref[...] += part_sum


@jax.jit
def kernel(z, embedding):
    B, T, D = z.shape
    n_tok = B * T
    zt = jnp.transpose(z, (0, 2, 1)).reshape(B * D, T)
    grid = B // HALVES

    zqt, idx3, loss_sum = pl.pallas_call(
        _vq_kernel,
        grid=(grid,),
        in_specs=[
            pl.BlockSpec((HALVES * D, BLOCK_T), lambda i: (i, 0)),
            pl.BlockSpec((N_EMBED, D), lambda i: (0, 0)),
        ],
        out_specs=[
            pl.BlockSpec((HALVES * D, BLOCK_T), lambda i: (i, 0)),
            pl.BlockSpec((1, HALVES, BLOCK_T), lambda i: (i, 0, 0)),
            pl.BlockSpec((1, 1), lambda i: (0, 0)),
        ],
        out_shape=[
            jax.ShapeDtypeStruct((B * D, T), jnp.float32),
            jax.ShapeDtypeStruct((grid, HALVES, BLOCK_T), jnp.int32),
            jax.ShapeDtypeStruct((1, 1), jnp.float32),
        ],
        scratch_shapes=[
            pltpu.VMEM((N_EMBED, EMBED_DIM + 1), jnp.float32),
            pltpu.VMEM((EMBED_DIM, N_EMBED), jnp.float32),
        ],
    )(zt, embedding)

    z_q = jnp.transpose(zqt.reshape(B, D, T), (0, 2, 1))
    encoding_indices = idx3.reshape(B, T)
    loss = 2.0 * loss_sum[0, 0] / (n_tok * D)
    return z_q, loss, encoding_indices


# speedup vs baseline: 6.0948x; 1.0679x over previous
import jax
import jax.numpy as jnp
from jax.experimental import pallas as pl
from jax.experimental.pallas import tpu as pltpu

EMBED_DIM = 32
N_EMBED = 512
BLOCK_T = 1024
HALVES = 8


def _vq_half(zbt, aug, ent):
    ss = jnp.sum(zbt * zbt, axis=0, keepdims=True)
    zinv = jax.lax.rsqrt(jnp.maximum(ss, 1e-24))
    znt = zbt * zinv
    zna = jnp.concatenate(
        [znt, jnp.ones((1, BLOCK_T), jnp.float32)], axis=0
    )

    d2 = jnp.dot(aug, zna, preferred_element_type=jnp.float32)

    m = jnp.min(d2, axis=0, keepdims=True)
    rowf = jax.lax.broadcasted_iota(jnp.int32, d2.shape, 0).astype(
        jnp.float32
    )
    idxf = jnp.min(
        jnp.where(d2 == m, rowf, float(N_EMBED)), axis=0, keepdims=True
    )

    onehot = (rowf == idxf).astype(jnp.float32)
    zqt = jnp.dot(ent, onehot, preferred_element_type=jnp.float32)

    part = jnp.sum(ss * (zinv * zinv) + m, axis=1, keepdims=True)
    return zbt + (zqt - zbt), idxf.astype(jnp.int32), part


def _vq_kernel(zt_ref, emb_ref, zqt_ref, idx_ref, loss_ref,
               aug_ref, ent_ref):
    i = pl.program_id(0)

    @pl.when(i == 0)
    def _():
        e = emb_ref[...]
        rn2 = jnp.sum(e * e, axis=1, keepdims=True)
        rinv = jax.lax.rsqrt(jnp.maximum(rn2, 1e-24))
        en = e * rinv
        en2c = rn2 * (rinv * rinv)
        aug_ref[...] = jnp.concatenate([en * -2.0, en2c], axis=1)
        ent_ref[...] = jnp.swapaxes(en, 0, 1)

    aug = aug_ref[...]
    ent = ent_ref[...]
    zbt = zt_ref[...]

    part_sum = jnp.zeros((1, 1), jnp.float32)
    zq_halves = []
    idx_halves = []
    for h in range(HALVES):
        zq_h, idx_h, part = _vq_half(
            zbt[h * EMBED_DIM:(h + 1) * EMBED_DIM, :], aug, ent
        )
        zq_halves.append(zq_h)
        idx_halves.append(idx_h.reshape(1, 1, BLOCK_T))
        part_sum = part_sum + part

    zqt_ref[...] = jnp.concatenate(zq_halves, axis=0)
    idx_ref[...] = jnp.concatenate(idx_halves, axis=1)

    @pl.when(i == 0)
    def _():
        loss_ref[...] = jnp.zeros((1, 1), jnp.float32)

    loss_ref[...] += part_sum


@jax.jit
def kernel(z, embedding):
    B, T, D = z.shape
    n_tok = B * T
    zt = jnp.transpose(z, (0, 2, 1)).reshape(B * D, T)
    grid = B // HALVES

    zqt, idx3, loss_sum = pl.pallas_call(
        _vq_kernel,
        grid=(grid,),
        in_specs=[
            pl.BlockSpec((HALVES * D, BLOCK_T), lambda i: (i, 0)),
            pl.BlockSpec((N_EMBED, D), lambda i: (0, 0)),
        ],
        out_specs=[
            pl.BlockSpec((HALVES * D, BLOCK_T), lambda i: (i, 0)),
            pl.BlockSpec((1, HALVES, BLOCK_T), lambda i: (i, 0, 0)),
            pl.BlockSpec((1, 1), lambda i: (0, 0)),
        ],
        out_shape=[
            jax.ShapeDtypeStruct((B * D, T), jnp.float32),
            jax.ShapeDtypeStruct((grid, HALVES, BLOCK_T), jnp.int32),
            jax.ShapeDtypeStruct((1, 1), jnp.float32),
        ],
        scratch_shapes=[
            pltpu.VMEM((N_EMBED, EMBED_DIM + 1), jnp.float32),
            pltpu.VMEM((EMBED_DIM, N_EMBED), jnp.float32),
        ],
    )(zt, embedding)

    z_q = jnp.transpose(zqt.reshape(B, D, T), (0, 2, 1))
    encoding_indices = idx3.reshape(B, T)
    loss = 2.0 * loss_sum[0, 0] / (n_tok * D)
    return z_q, loss, encoding_indices


# speedup vs baseline: 6.1420x; 1.0077x over previous
import jax
import jax.numpy as jnp
from jax.experimental import pallas as pl
from jax.experimental.pallas import tpu as pltpu

EMBED_DIM = 32
N_EMBED = 512
BLOCK_T = 1024
HALVES = 16


def _vq_half(zbt, aug, ent):
    ss = jnp.sum(zbt * zbt, axis=0, keepdims=True)
    zinv = jax.lax.rsqrt(jnp.maximum(ss, 1e-24))
    znt = zbt * zinv
    zna = jnp.concatenate(
        [znt, jnp.ones((1, BLOCK_T), jnp.float32)], axis=0
    )

    d2 = jnp.dot(aug, zna, preferred_element_type=jnp.float32)

    m = jnp.min(d2, axis=0, keepdims=True)
    rowf = jax.lax.broadcasted_iota(jnp.int32, d2.shape, 0).astype(
        jnp.float32
    )
    idxf = jnp.min(
        jnp.where(d2 == m, rowf, float(N_EMBED)), axis=0, keepdims=True
    )

    onehot = (rowf == idxf).astype(jnp.float32)
    zqt = jnp.dot(ent, onehot, preferred_element_type=jnp.float32)

    part = jnp.sum(ss * (zinv * zinv) + m, axis=1, keepdims=True)
    return zbt + (zqt - zbt), idxf.astype(jnp.int32), part


def _vq_kernel(zt_ref, emb_ref, zqt_ref, idx_ref, loss_ref,
               aug_ref, ent_ref):
    i = pl.program_id(0)

    @pl.when(i == 0)
    def _():
        e = emb_ref[...]
        rn2 = jnp.sum(e * e, axis=1, keepdims=True)
        rinv = jax.lax.rsqrt(jnp.maximum(rn2, 1e-24))
        en = e * rinv
        en2c = rn2 * (rinv * rinv)
        aug_ref[...] = jnp.concatenate([en * -2.0, en2c], axis=1)
        ent_ref[...] = jnp.swapaxes(en, 0, 1)

    aug = aug_ref[...]
    ent = ent_ref[...]
    zbt = zt_ref[...]

    part_sum = jnp.zeros((1, 1), jnp.float32)
    zq_halves = []
    idx_halves = []
    for h in range(HALVES):
        zq_h, idx_h, part = _vq_half(
            zbt[h * EMBED_DIM:(h + 1) * EMBED_DIM, :], aug, ent
        )
        zq_halves.append(zq_h)
        idx_halves.append(idx_h.reshape(1, 1, BLOCK_T))
        part_sum = part_sum + part

    zqt_ref[...] = jnp.concatenate(zq_halves, axis=0)
    idx_ref[...] = jnp.concatenate(idx_halves, axis=1)

    @pl.when(i == 0)
    def _():
        loss_ref[...] = jnp.zeros((1, 1), jnp.float32)

    loss_ref[...] += part_sum


@jax.jit
def kernel(z, embedding):
    B, T, D = z.shape
    n_tok = B * T
    zt = jnp.transpose(z, (0, 2, 1)).reshape(B * D, T)
    grid = B // HALVES

    zqt, idx3, loss_sum = pl.pallas_call(
        _vq_kernel,
        grid=(grid,),
        in_specs=[
            pl.BlockSpec((HALVES * D, BLOCK_T), lambda i: (i, 0)),
            pl.BlockSpec((N_EMBED, D), lambda i: (0, 0)),
        ],
        out_specs=[
            pl.BlockSpec((HALVES * D, BLOCK_T), lambda i: (i, 0)),
            pl.BlockSpec((1, HALVES, BLOCK_T), lambda i: (i, 0, 0)),
            pl.BlockSpec((1, 1), lambda i: (0, 0)),
        ],
        out_shape=[
            jax.ShapeDtypeStruct((B * D, T), jnp.float32),
            jax.ShapeDtypeStruct((grid, HALVES, BLOCK_T), jnp.int32),
            jax.ShapeDtypeStruct((1, 1), jnp.float32),
        ],
        scratch_shapes=[
            pltpu.VMEM((N_EMBED, EMBED_DIM + 1), jnp.float32),
            pltpu.VMEM((EMBED_DIM, N_EMBED), jnp.float32),
        ],
    )(zt, embedding)

    z_q = jnp.transpose(zqt.reshape(B, D, T), (0, 2, 1))
    encoding_indices = idx3.reshape(B, T)
    loss = 2.0 * loss_sum[0, 0] / (n_tok * D)
    return z_q, loss, encoding_indices


# speedup vs baseline: 7.1625x; 1.1662x over previous
import jax
import jax.numpy as jnp
from jax.experimental import pallas as pl
from jax.experimental.pallas import tpu as pltpu

EMBED_DIM = 32
N_EMBED = 512
BLOCK_T = 1024
HALVES = 16


def _vq_half(zbt, aug, ent):
    ss = jnp.sum(zbt * zbt, axis=0, keepdims=True)
    zinv = jax.lax.rsqrt(jnp.maximum(ss, 1e-24))
    znt = zbt * zinv
    zna = jnp.concatenate(
        [znt, jnp.ones((1, BLOCK_T), jnp.float32)], axis=0
    )

    d2 = jnp.dot(aug, zna, preferred_element_type=jnp.float32)

    m = jnp.min(d2, axis=0, keepdims=True)
    idxi = jnp.argmin(d2, axis=0)[None, :]
    rowi = jax.lax.broadcasted_iota(jnp.int32, d2.shape, 0)

    onehot = (rowi == idxi).astype(jnp.float32)
    zqt = jnp.dot(ent, onehot, preferred_element_type=jnp.float32)

    part = jnp.sum(ss * (zinv * zinv) + m, axis=1, keepdims=True)
    return zbt + (zqt - zbt), idxi, part


def _vq_kernel(zt_ref, emb_ref, zqt_ref, idx_ref, loss_ref,
               aug_ref, ent_ref):
    i = pl.program_id(0)

    @pl.when(i == 0)
    def _():
        e = emb_ref[...]
        rn2 = jnp.sum(e * e, axis=1, keepdims=True)
        rinv = jax.lax.rsqrt(jnp.maximum(rn2, 1e-24))
        en = e * rinv
        en2c = rn2 * (rinv * rinv)
        aug_ref[...] = jnp.concatenate([en * -2.0, en2c], axis=1)
        ent_ref[...] = jnp.swapaxes(en, 0, 1)

    aug = aug_ref[...]
    ent = ent_ref[...]
    zbt = zt_ref[...]

    part_sum = jnp.zeros((1, 1), jnp.float32)
    zq_halves = []
    idx_halves = []
    for h in range(HALVES):
        zq_h, idx_h, part = _vq_half(
            zbt[h * EMBED_DIM:(h + 1) * EMBED_DIM, :], aug, ent
        )
        zq_halves.append(zq_h)
        idx_halves.append(idx_h.reshape(1, 1, BLOCK_T))
        part_sum = part_sum + part

    zqt_ref[...] = jnp.concatenate(zq_halves, axis=0)
    idx_ref[...] = jnp.concatenate(idx_halves, axis=1)

    @pl.when(i == 0)
    def _():
        loss_ref[...] = jnp.zeros((1, 1), jnp.float32)

    loss_ref[...] += part_sum


@jax.jit
def kernel(z, embedding):
    B, T, D = z.shape
    n_tok = B * T
    zt = jnp.transpose(z, (0, 2, 1)).reshape(B * D, T)
    grid = B // HALVES

    zqt, idx3, loss_sum = pl.pallas_call(
        _vq_kernel,
        grid=(grid,),
        in_specs=[
            pl.BlockSpec((HALVES * D, BLOCK_T), lambda i: (i, 0)),
            pl.BlockSpec((N_EMBED, D), lambda i: (0, 0)),
        ],
        out_specs=[
            pl.BlockSpec((HALVES * D, BLOCK_T), lambda i: (i, 0)),
            pl.BlockSpec((1, HALVES, BLOCK_T), lambda i: (i, 0, 0)),
            pl.BlockSpec((1, 1), lambda i: (0, 0)),
        ],
        out_shape=[
            jax.ShapeDtypeStruct((B * D, T), jnp.float32),
            jax.ShapeDtypeStruct((grid, HALVES, BLOCK_T), jnp.int32),
            jax.ShapeDtypeStruct((1, 1), jnp.float32),
        ],
        scratch_shapes=[
            pltpu.VMEM((N_EMBED, EMBED_DIM + 1), jnp.float32),
            pltpu.VMEM((EMBED_DIM, N_EMBED), jnp.float32),
        ],
    )(zt, embedding)

    z_q = jnp.transpose(zqt.reshape(B, D, T), (0, 2, 1))
    encoding_indices = idx3.reshape(B, T)
    loss = 2.0 * loss_sum[0, 0] / (n_tok * D)
    return z_q, loss, encoding_indices
